# Initial kernel scaffold; baseline (speedup 1.0000x reference)
import jax
import jax.numpy as jnp
from jax.experimental import pallas as pl


def kernel(h, edge_index, params):
    raise NotImplementedError("write your pallas kernel here")


# speedup vs baseline: 6.6628x; 6.6628x over previous
import functools

import jax
import jax.numpy as jnp
from jax import lax
from jax.experimental import pallas as pl
from jax.experimental.pallas import tpu as pltpu
from jax.experimental.pallas import tpu_sc as plsc

N = 10000
D = 128
E = 320000
L = 4

NC = 2
NS = 16
NW = NC * NS
EPW = E // NW
CHUNK = 80
NCHUNK = EPW // CHUNK
SLAB = 624

BR = 2000
GRID = N // BR


def _segsum_body(x_hbm, src_hbm, dst_hbm, out_hbm,
                 src_v, dst_v, rows_v, acc, sem):
    c = lax.axis_index("c")
    s = lax.axis_index("s")
    wid = s * NC + c

    pltpu.sync_copy(src_hbm.at[wid], src_v)
    pltpu.sync_copy(dst_hbm.at[wid], dst_v)

    def _zrow(r, carry):
        for j in range(D // 16):
            rows_v[r, pl.ds(j * 16, 16)] = jnp.zeros((16,), jnp.float32)
        return carry

    lax.fori_loop(0, CHUNK, _zrow, 0)
    base = s * SLAB
    for q in range(SLAB // CHUNK):
        pltpu.sync_copy(rows_v, acc.at[pl.ds(base + q * CHUNK, CHUNK)])
    rem = SLAB % CHUNK
    if rem:
        pltpu.sync_copy(rows_v.at[pl.ds(0, rem)],
                        acc.at[pl.ds(base + SLAB - rem, rem)])

    @pl.when(s == NS - 1)
    def _zero_tail():
        pltpu.sync_copy(rows_v.at[pl.ds(0, N - NS * SLAB)],
                        acc.at[pl.ds(NS * SLAB, N - NS * SLAB)])

    plsc.subcore_barrier()

    def body(j, carry):
        pltpu.async_copy(x_hbm.at[src_v.at[j]], rows_v, sem).wait()
        pltpu.sync_copy(rows_v, acc.at[dst_v.at[j]], add=True)
        return carry

    lax.fori_loop(0, NCHUNK, body, 0)
    plsc.subcore_barrier()

    pltpu.sync_copy(acc.at[pl.ds(base, SLAB)], out_hbm.at[c, pl.ds(base, SLAB)])

    @pl.when(s == NS - 1)
    def _out_tail():
        pltpu.sync_copy(acc.at[pl.ds(NS * SLAB, N - NS * SLAB)],
                        out_hbm.at[c, pl.ds(NS * SLAB, N - NS * SLAB)])


@functools.cache
def _segsum_call():
    return pl.kernel(
        _segsum_body,
        out_type=jax.ShapeDtypeStruct((NC, N, D), jnp.float32),
        mesh=plsc.VectorSubcoreMesh(
            core_axis_name="c", subcore_axis_name="s",
            num_cores=NC, num_subcores=NS),
        scratch_types=[
            pltpu.VMEM((NCHUNK, CHUNK), jnp.int32),
            pltpu.VMEM((NCHUNK, CHUNK), jnp.int32),
            pltpu.VMEM((CHUNK, D), jnp.float32),
            pltpu.VMEM_SHARED((N, D), jnp.float32),
            pltpu.SemaphoreType.DMA,
        ],
    )


def _ln(t, g, b):
    m = jnp.mean(t, axis=-1, keepdims=True)
    v = jnp.mean((t - m) * (t - m), axis=-1, keepdims=True)
    return (t - m) * lax.rsqrt(v + 1e-5) * g + b


def _inln_body(h_ref, g_ref, b_ref, o_ref):
    o_ref[...] = _ln(h_ref[...], g_ref[...], b_ref[...])


def _mlp_compute(x, parts_ref, w1_ref, b1_ref, g1_ref, be1_ref,
                 w2_ref, b2_ref, g2_ref, be2_ref, gn_ref, bn_ref, residual):
    z = x + parts_ref[0] + parts_ref[1]
    t = jnp.dot(z, w1_ref[...], preferred_element_type=jnp.float32) + b1_ref[...]
    t = jnp.maximum(_ln(t, g1_ref[...], be1_ref[...]), 0.0)
    t = jnp.dot(t, w2_ref[...], preferred_element_type=jnp.float32) + b2_ref[...]
    t = jnp.maximum(_ln(t, g2_ref[...], be2_ref[...]), 0.0)
    t = jnp.maximum(_ln(t, gn_ref[...], bn_ref[...]), 0.0)
    if residual:
        t = t + x
    return t


def _mlp_body(residual, x_ref, parts_ref, w1_ref, b1_ref, g1_ref, be1_ref,
              w2_ref, b2_ref, g2_ref, be2_ref, gn_ref, bn_ref, o_ref):
    o_ref[...] = _mlp_compute(
        x_ref[...], parts_ref, w1_ref, b1_ref, g1_ref, be1_ref,
        w2_ref, b2_ref, g2_ref, be2_ref, gn_ref, bn_ref, residual)


def _mlp_final_body(x_ref, parts_ref, w1_ref, b1_ref, g1_ref, be1_ref,
                    w2_ref, b2_ref, g2_ref, be2_ref, gn_ref, bn_ref, o_ref):
    t = _mlp_compute(
        x_ref[...], parts_ref, w1_ref, b1_ref, g1_ref, be1_ref,
        w2_ref, b2_ref, g2_ref, be2_ref, gn_ref, bn_ref, True)

    @pl.when(pl.program_id(0) == 0)
    def _init():
        o_ref[...] = jnp.zeros_like(o_ref)

    o_ref[...] += jnp.sum(t, axis=0, keepdims=True) * (1.0 / N)


def _row_spec():
    return pl.BlockSpec((BR, D), lambda i: (i, 0))


def _mlp_in_specs():
    full = lambda shape: pl.BlockSpec(shape, lambda i: tuple(0 for _ in shape))
    return [
        _row_spec(),
        pl.BlockSpec((NC, BR, D), lambda i: (0, i, 0)),
        full((D, D)), full((1, D)), full((1, D)), full((1, D)),
        full((D, D)), full((1, D)), full((1, D)), full((1, D)),
        full((1, D)), full((1, D)),
    ]


@functools.cache
def _inln_call():
    full = lambda shape: pl.BlockSpec(shape, lambda i: tuple(0 for _ in shape))
    return pl.pallas_call(
        _inln_body,
        grid=(GRID,),
        in_specs=[_row_spec(), full((1, D)), full((1, D))],
        out_specs=_row_spec(),
        out_shape=jax.ShapeDtypeStruct((N, D), jnp.float32),
    )


@functools.cache
def _mlp_call(residual):
    return pl.pallas_call(
        functools.partial(_mlp_body, residual),
        grid=(GRID,),
        in_specs=_mlp_in_specs(),
        out_specs=_row_spec(),
        out_shape=jax.ShapeDtypeStruct((N, D), jnp.float32),
    )


@functools.cache
def _mlp_final_call():
    return pl.pallas_call(
        _mlp_final_body,
        grid=(GRID,),
        in_specs=_mlp_in_specs(),
        out_specs=pl.BlockSpec((1, D), lambda i: (0, 0)),
        out_shape=jax.ShapeDtypeStruct((1, D), jnp.float32),
    )


def kernel(h, edge_index, params):
    src3 = edge_index[0].reshape(NW, NCHUNK, CHUNK)
    dst3 = edge_index[1].reshape(NW, NCHUNK, CHUNK)
    v = lambda a: a.reshape(1, D)

    x = _inln_call()(h, v(params["in_g"]), v(params["in_b"]))
    out = None
    for i, p in enumerate(params["layers"]):
        parts = _segsum_call()(x, src3, dst3)
        args = (x, parts,
                p["w1"], v(p["b1"]), v(p["ln1_g"]), v(p["ln1_b"]),
                p["w2"], v(p["b2"]), v(p["ln2_g"]), v(p["ln2_b"]),
                v(p["n_g"]), v(p["n_b"]))
        if i < L - 1:
            x = _mlp_call(i > 0)(*args)
        else:
            out = _mlp_final_call()(*args)
    return out


# speedup vs baseline: 10.8540x; 1.6291x over previous
import functools

import jax
import jax.numpy as jnp
from jax import lax
from jax.experimental import pallas as pl
from jax.experimental.pallas import tpu as pltpu
from jax.experimental.pallas import tpu_sc as plsc

N = 10000
D = 128
E = 320000
L = 4

NC = 2
NS = 16
NW = NC * NS
EPW = E // NW
CHUNK = 80
NCHUNK = EPW // CHUNK
SLAB = 624

BR = 2000
GRID = N // BR


def _segsum_body(x_hbm, src_hbm, dst_hbm, out_hbm,
                 src_v, dst_v, rows0, rows1, acc, sem0, sem1):
    c = lax.axis_index("c")
    s = lax.axis_index("s")
    wid = s * NC + c

    pltpu.sync_copy(src_hbm.at[pl.ds(wid * EPW, EPW)], src_v)
    pltpu.sync_copy(dst_hbm.at[wid], dst_v)

    def _zrow(r, carry):
        for j in range(D // 16):
            rows0[r, pl.ds(j * 16, 16)] = jnp.zeros((16,), jnp.float32)
        return carry

    lax.fori_loop(0, CHUNK, _zrow, 0)
    base = s * SLAB
    for q in range(SLAB // CHUNK):
        pltpu.sync_copy(rows0, acc.at[pl.ds(base + q * CHUNK, CHUNK)])
    rem = SLAB % CHUNK
    if rem:
        pltpu.sync_copy(rows0.at[pl.ds(0, rem)],
                        acc.at[pl.ds(base + SLAB - rem, rem)])

    @pl.when(s == NS - 1)
    def _zero_tail():
        pltpu.sync_copy(rows0.at[pl.ds(0, N - NS * SLAB)],
                        acc.at[pl.ds(NS * SLAB, N - NS * SLAB)])

    plsc.subcore_barrier()

    def _gather(chunk, buf, sem):
        off = pl.multiple_of(chunk * CHUNK, 8)
        pltpu.async_copy(x_hbm.at[src_v.at[pl.ds(off, CHUNK)]], buf, sem)

    def _gwait(buf, sem):
        pltpu.make_async_copy(
            x_hbm.at[src_v.at[pl.ds(0, CHUNK)]], buf, sem).wait()

    def _scatter(chunk, buf):
        pltpu.sync_copy(buf, acc.at[dst_v.at[chunk]], add=True)

    _gather(0, rows0, sem0)

    def body(k, carry):
        a = 2 * k
        _gather(a + 1, rows1, sem1)
        _gwait(rows0, sem0)
        _scatter(a, rows0)
        _gather(a + 2, rows0, sem0)
        _gwait(rows1, sem1)
        _scatter(a + 1, rows1)
        return carry

    lax.fori_loop(0, (NCHUNK - 1) // 2, body, 0)
    _gwait(rows0, sem0)
    _scatter(NCHUNK - 1, rows0)
    plsc.subcore_barrier()

    pltpu.sync_copy(acc.at[pl.ds(base, SLAB)], out_hbm.at[c, pl.ds(base, SLAB)])

    @pl.when(s == NS - 1)
    def _out_tail():
        pltpu.sync_copy(acc.at[pl.ds(NS * SLAB, N - NS * SLAB)],
                        out_hbm.at[c, pl.ds(NS * SLAB, N - NS * SLAB)])


@functools.cache
def _segsum_call():
    return pl.kernel(
        _segsum_body,
        out_type=jax.ShapeDtypeStruct((NC, N, D), jnp.float32),
        mesh=plsc.VectorSubcoreMesh(
            core_axis_name="c", subcore_axis_name="s",
            num_cores=NC, num_subcores=NS),
        scratch_types=[
            pltpu.VMEM((EPW,), jnp.int32),
            pltpu.VMEM((NCHUNK, CHUNK), jnp.int32),
            pltpu.VMEM((CHUNK, D), jnp.float32),
            pltpu.VMEM((CHUNK, D), jnp.float32),
            pltpu.VMEM_SHARED((N, D), jnp.float32),
            pltpu.SemaphoreType.DMA,
            pltpu.SemaphoreType.DMA,
        ],
    )


def _ln(t, g, b):
    m = jnp.mean(t, axis=-1, keepdims=True)
    v = jnp.mean((t - m) * (t - m), axis=-1, keepdims=True)
    return (t - m) * lax.rsqrt(v + 1e-5) * g + b


def _inln_body(h_ref, g_ref, b_ref, o_ref):
    o_ref[...] = _ln(h_ref[...], g_ref[...], b_ref[...])


def _mlp_compute(x, parts_ref, w1_ref, b1_ref, g1_ref, be1_ref,
                 w2_ref, b2_ref, g2_ref, be2_ref, gn_ref, bn_ref, residual):
    z = x + parts_ref[0] + parts_ref[1]
    t = jnp.dot(z, w1_ref[...], preferred_element_type=jnp.float32) + b1_ref[...]
    t = jnp.maximum(_ln(t, g1_ref[...], be1_ref[...]), 0.0)
    t = jnp.dot(t, w2_ref[...], preferred_element_type=jnp.float32) + b2_ref[...]
    t = jnp.maximum(_ln(t, g2_ref[...], be2_ref[...]), 0.0)
    t = jnp.maximum(_ln(t, gn_ref[...], bn_ref[...]), 0.0)
    if residual:
        t = t + x
    return t


def _mlp_body(residual, x_ref, parts_ref, w1_ref, b1_ref, g1_ref, be1_ref,
              w2_ref, b2_ref, g2_ref, be2_ref, gn_ref, bn_ref, o_ref):
    o_ref[...] = _mlp_compute(
        x_ref[...], parts_ref, w1_ref, b1_ref, g1_ref, be1_ref,
        w2_ref, b2_ref, g2_ref, be2_ref, gn_ref, bn_ref, residual)


def _mlp_final_body(x_ref, parts_ref, w1_ref, b1_ref, g1_ref, be1_ref,
                    w2_ref, b2_ref, g2_ref, be2_ref, gn_ref, bn_ref, o_ref):
    t = _mlp_compute(
        x_ref[...], parts_ref, w1_ref, b1_ref, g1_ref, be1_ref,
        w2_ref, b2_ref, g2_ref, be2_ref, gn_ref, bn_ref, True)

    @pl.when(pl.program_id(0) == 0)
    def _init():
        o_ref[...] = jnp.zeros_like(o_ref)

    o_ref[...] += jnp.sum(t, axis=0, keepdims=True) * (1.0 / N)


def _row_spec():
    return pl.BlockSpec((BR, D), lambda i: (i, 0))


def _mlp_in_specs():
    full = lambda shape: pl.BlockSpec(shape, lambda i: tuple(0 for _ in shape))
    return [
        _row_spec(),
        pl.BlockSpec((NC, BR, D), lambda i: (0, i, 0)),
        full((D, D)), full((1, D)), full((1, D)), full((1, D)),
        full((D, D)), full((1, D)), full((1, D)), full((1, D)),
        full((1, D)), full((1, D)),
    ]


@functools.cache
def _inln_call():
    full = lambda shape: pl.BlockSpec(shape, lambda i: tuple(0 for _ in shape))
    return pl.pallas_call(
        _inln_body,
        grid=(GRID,),
        in_specs=[_row_spec(), full((1, D)), full((1, D))],
        out_specs=_row_spec(),
        out_shape=jax.ShapeDtypeStruct((N, D), jnp.float32),
    )


@functools.cache
def _mlp_call(residual):
    return pl.pallas_call(
        functools.partial(_mlp_body, residual),
        grid=(GRID,),
        in_specs=_mlp_in_specs(),
        out_specs=_row_spec(),
        out_shape=jax.ShapeDtypeStruct((N, D), jnp.float32),
    )


@functools.cache
def _mlp_final_call():
    return pl.pallas_call(
        _mlp_final_body,
        grid=(GRID,),
        in_specs=_mlp_in_specs(),
        out_specs=pl.BlockSpec((1, D), lambda i: (0, 0)),
        out_shape=jax.ShapeDtypeStruct((1, D), jnp.float32),
    )


def kernel(h, edge_index, params):
    src3 = edge_index[0]
    dst3 = edge_index[1].reshape(NW, NCHUNK, CHUNK)
    v = lambda a: a.reshape(1, D)

    x = _inln_call()(h, v(params["in_g"]), v(params["in_b"]))
    out = None
    for i, p in enumerate(params["layers"]):
        parts = _segsum_call()(x, src3, dst3)
        args = (x, parts,
                p["w1"], v(p["b1"]), v(p["ln1_g"]), v(p["ln1_b"]),
                p["w2"], v(p["b2"]), v(p["ln2_g"]), v(p["ln2_b"]),
                v(p["n_g"]), v(p["n_b"]))
        if i < L - 1:
            x = _mlp_call(i > 0)(*args)
        else:
            out = _mlp_final_call()(*args)
    return out


# speedup vs baseline: 11.3217x; 1.0431x over previous
import functools

import jax
import jax.numpy as jnp
from jax import lax
from jax.experimental import pallas as pl
from jax.experimental.pallas import tpu as pltpu
from jax.experimental.pallas import tpu_sc as plsc

N = 10000
D = 128
E = 320000
L = 4

NC = 2
NS = 16
NW = NC * NS
EPW = E // NW
CHUNK = 80
NCHUNK = EPW // CHUNK
SLAB = 624

BR = 2000
GRID = N // BR


def _segsum_body(x_hbm, src_hbm, dst_hbm, out_hbm, *refs):
    (src_ring, dst_ring, rows0, rows1, rows2, rows3, acc,
     g0, g1, g2, g3, s0, s1, s2, s3,
     i0, i1, i2, i3, i4, i5, i6, i7) = refs
    rows = (rows0, rows1, rows2, rows3)
    gsem = (g0, g1, g2, g3)
    ssem = (s0, s1, s2, s3)
    isem = (i0, i1, i2, i3, i4, i5, i6, i7)

    c = lax.axis_index("c")
    s = lax.axis_index("s")
    wid = s * NC + c
    ebase = wid * EPW

    def _zrow(r, carry):
        for j in range(D // 16):
            rows0[r, pl.ds(j * 16, 16)] = jnp.zeros((16,), jnp.float32)
        return carry

    lax.fori_loop(0, CHUNK, _zrow, 0)
    base = s * SLAB
    for q in range(SLAB // CHUNK):
        pltpu.sync_copy(rows0, acc.at[pl.ds(base + q * CHUNK, CHUNK)])
    rem = SLAB % CHUNK
    if rem:
        pltpu.sync_copy(rows0.at[pl.ds(0, rem)],
                        acc.at[pl.ds(base + SLAB - rem, rem)])

    @pl.when(s == NS - 1)
    def _zero_tail():
        pltpu.sync_copy(rows0.at[pl.ds(0, N - NS * SLAB)],
                        acc.at[pl.ds(NS * SLAB, N - NS * SLAB)])

    plsc.subcore_barrier()

    def _ifire(j, i):
        off = pl.multiple_of(ebase + j * CHUNK, 8)
        pltpu.async_copy(src_hbm.at[pl.ds(off, CHUNK)], src_ring.at[i],
                         isem[i])
        pltpu.async_copy(dst_hbm.at[pl.ds(off, CHUNK)], dst_ring.at[i],
                         isem[i])

    def _iwait(i):
        pltpu.make_async_copy(src_hbm.at[pl.ds(0, CHUNK)], src_ring.at[i],
                              isem[i]).wait()
        pltpu.make_async_copy(dst_hbm.at[pl.ds(0, CHUNK)], dst_ring.at[i],
                              isem[i]).wait()

    def _gfire(r, i):
        pltpu.async_copy(x_hbm.at[src_ring.at[i]], rows[r], gsem[r])

    def _gwait(r):
        pltpu.make_async_copy(x_hbm.at[src_ring.at[0]], rows[r],
                              gsem[r]).wait()

    def _sfire(r, i):
        pltpu.async_copy(rows[r], acc.at[dst_ring.at[i]], ssem[r], add=True)

    def _swait(r):
        pltpu.make_async_copy(rows[r], acc.at[dst_ring.at[0]],
                              ssem[r]).wait()

    def _step(w, b, first):
        j = w + b
        r = b % 4
        _gwait(r)
        _sfire(r, b % 8)
        rr = (b + 2) % 4
        ii = (b + 2) % 8
        if not (first and b < 2):
            _swait(rr)
        _iwait(ii)
        _gfire(rr, ii)
        jj = j + 6

        @pl.when(jj < NCHUNK)
        def _pref():
            _ifire(jj, (b + 6) % 8)

    for j in range(6):
        _ifire(j, j)
    _iwait(0)
    _gfire(0, 0)
    _iwait(1)
    _gfire(1, 1)

    for b in range(8):
        _step(0, b, True)

    def _window(k, carry):
        for b in range(8):
            _step(8 * k, b, False)
        return carry

    lax.fori_loop(1, NCHUNK // 8, _window, 0)

    for b in range(NCHUNK % 8):
        j = 8 * (NCHUNK // 8) + b
        r = b % 4
        _gwait(r)
        _sfire(r, b % 8)
        jj = j + 2
        if jj < NCHUNK:
            _swait(jj % 4)
            _iwait(jj % 8)
            _gfire(jj % 4, jj % 8)
    for r in range(4):
        _swait(r)
    plsc.subcore_barrier()

    pltpu.sync_copy(acc.at[pl.ds(base, SLAB)], out_hbm.at[c, pl.ds(base, SLAB)])

    @pl.when(s == NS - 1)
    def _out_tail():
        pltpu.sync_copy(acc.at[pl.ds(NS * SLAB, N - NS * SLAB)],
                        out_hbm.at[c, pl.ds(NS * SLAB, N - NS * SLAB)])


@functools.cache
def _segsum_call():
    return pl.kernel(
        _segsum_body,
        out_type=jax.ShapeDtypeStruct((NC, N, D), jnp.float32),
        mesh=plsc.VectorSubcoreMesh(
            core_axis_name="c", subcore_axis_name="s",
            num_cores=NC, num_subcores=NS),
        scratch_types=(
            [pltpu.VMEM((8, CHUNK), jnp.int32),
             pltpu.VMEM((8, CHUNK), jnp.int32)]
            + [pltpu.VMEM((CHUNK, D), jnp.float32) for _ in range(4)]
            + [pltpu.VMEM_SHARED((N, D), jnp.float32)]
            + [pltpu.SemaphoreType.DMA for _ in range(16)]
        ),
    )


def _ln(t, g, b):
    m = jnp.mean(t, axis=-1, keepdims=True)
    v = jnp.mean((t - m) * (t - m), axis=-1, keepdims=True)
    return (t - m) * lax.rsqrt(v + 1e-5) * g + b


def _inln_body(h_ref, g_ref, b_ref, o_ref):
    o_ref[...] = _ln(h_ref[...], g_ref[...], b_ref[...])


def _mlp_compute(x, parts_ref, w1_ref, b1_ref, g1_ref, be1_ref,
                 w2_ref, b2_ref, g2_ref, be2_ref, gn_ref, bn_ref, residual):
    z = x + parts_ref[0] + parts_ref[1]
    t = jnp.dot(z, w1_ref[...], preferred_element_type=jnp.float32) + b1_ref[...]
    t = jnp.maximum(_ln(t, g1_ref[...], be1_ref[...]), 0.0)
    t = jnp.dot(t, w2_ref[...], preferred_element_type=jnp.float32) + b2_ref[...]
    t = jnp.maximum(_ln(t, g2_ref[...], be2_ref[...]), 0.0)
    t = jnp.maximum(_ln(t, gn_ref[...], bn_ref[...]), 0.0)
    if residual:
        t = t + x
    return t


def _mlp_body(residual, x_ref, parts_ref, w1_ref, b1_ref, g1_ref, be1_ref,
              w2_ref, b2_ref, g2_ref, be2_ref, gn_ref, bn_ref, o_ref):
    o_ref[...] = _mlp_compute(
        x_ref[...], parts_ref, w1_ref, b1_ref, g1_ref, be1_ref,
        w2_ref, b2_ref, g2_ref, be2_ref, gn_ref, bn_ref, residual)


def _mlp_final_body(x_ref, parts_ref, w1_ref, b1_ref, g1_ref, be1_ref,
                    w2_ref, b2_ref, g2_ref, be2_ref, gn_ref, bn_ref, o_ref):
    t = _mlp_compute(
        x_ref[...], parts_ref, w1_ref, b1_ref, g1_ref, be1_ref,
        w2_ref, b2_ref, g2_ref, be2_ref, gn_ref, bn_ref, True)

    @pl.when(pl.program_id(0) == 0)
    def _init():
        o_ref[...] = jnp.zeros_like(o_ref)

    o_ref[...] += jnp.sum(t, axis=0, keepdims=True) * (1.0 / N)


def _row_spec():
    return pl.BlockSpec((BR, D), lambda i: (i, 0))


def _mlp_in_specs():
    full = lambda shape: pl.BlockSpec(shape, lambda i: tuple(0 for _ in shape))
    return [
        _row_spec(),
        pl.BlockSpec((NC, BR, D), lambda i: (0, i, 0)),
        full((D, D)), full((1, D)), full((1, D)), full((1, D)),
        full((D, D)), full((1, D)), full((1, D)), full((1, D)),
        full((1, D)), full((1, D)),
    ]


@functools.cache
def _inln_call():
    full = lambda shape: pl.BlockSpec(shape, lambda i: tuple(0 for _ in shape))
    return pl.pallas_call(
        _inln_body,
        grid=(GRID,),
        in_specs=[_row_spec(), full((1, D)), full((1, D))],
        out_specs=_row_spec(),
        out_shape=jax.ShapeDtypeStruct((N, D), jnp.float32),
    )


@functools.cache
def _mlp_call(residual):
    return pl.pallas_call(
        functools.partial(_mlp_body, residual),
        grid=(GRID,),
        in_specs=_mlp_in_specs(),
        out_specs=_row_spec(),
        out_shape=jax.ShapeDtypeStruct((N, D), jnp.float32),
    )


@functools.cache
def _mlp_final_call():
    return pl.pallas_call(
        _mlp_final_body,
        grid=(GRID,),
        in_specs=_mlp_in_specs(),
        out_specs=pl.BlockSpec((1, D), lambda i: (0, 0)),
        out_shape=jax.ShapeDtypeStruct((1, D), jnp.float32),
    )


def kernel(h, edge_index, params):
    src3 = edge_index[0]
    dst3 = edge_index[1]
    v = lambda a: a.reshape(1, D)

    x = _inln_call()(h, v(params["in_g"]), v(params["in_b"]))
    out = None
    for i, p in enumerate(params["layers"]):
        parts = _segsum_call()(x, src3, dst3)
        args = (x, parts,
                p["w1"], v(p["b1"]), v(p["ln1_g"]), v(p["ln1_b"]),
                p["w2"], v(p["b2"]), v(p["ln2_g"]), v(p["ln2_b"]),
                v(p["n_g"]), v(p["n_b"]))
        if i < L - 1:
            x = _mlp_call(i > 0)(*args)
        else:
            out = _mlp_final_call()(*args)
    return out
